# Initial kernel scaffold; baseline (speedup 1.0000x reference)
import jax
import jax.numpy as jnp
from jax.experimental import pallas as pl


def kernel(X, h_V, edge_index, batch_id, params):
    raise NotImplementedError("write your pallas kernel here")


# speedup vs baseline: 3.4626x; 3.4626x over previous
import functools

import numpy as np
import jax
import jax.numpy as jnp
from jax import lax
from jax.experimental import pallas as pl
from jax.experimental.pallas import tpu as pltpu
from jax.experimental.pallas import tpu_sc as plsc

N = 10000
E = 160000
EP = 163840
H = 128
HEADS = 4
DH = H // HEADS
NG = 8
DV = H
HV_IN = 1024
GEO_N = 184
GEO_E = 450

BN = 2000
BE = 4096
GN = N // BN
GE = EP // BE

_BN_SCALE = float(1.0 / np.sqrt(1.0 + 1e-5))
_ISQ = float(1.0 / np.sqrt(DH))
_ISQ2 = float(1.0 / np.sqrt(2.0))

_F32 = jnp.float32


def _norm(x, axis=-1, eps=1e-12):
    n = jnp.linalg.norm(x, axis=axis, keepdims=True)
    return x / jnp.maximum(n, eps)


def _angle_feat(X, eps=1e-7):
    Xr = X[:, :3].reshape(3 * X.shape[0], 3)
    dX = Xr[1:] - Xr[:-1]
    U = _norm(dX)
    u_2, u_1, u_0 = U[:-2], U[1:-1], U[2:]
    n_2 = _norm(jnp.cross(u_2, u_1))
    n_1 = _norm(jnp.cross(u_1, u_0))
    cosD = jnp.clip(jnp.sum(n_2 * n_1, -1), -1 + eps, 1 - eps)
    D = jnp.sign(jnp.sum(u_2 * n_1, -1)) * jnp.arccos(cosD)
    D = jnp.pad(D, (1, 2)).reshape(-1, 3)
    dihedral = jnp.concatenate([jnp.cos(D), jnp.sin(D)], 1)
    cosB = jnp.clip(jnp.sum(u_2 * u_1, -1), -1 + eps, 1 - eps)
    B = jnp.pad(jnp.arccos(cosB), (1, 2)).reshape(-1, 3)
    bond = jnp.concatenate([jnp.cos(B), jnp.sin(B)], 1)
    return jnp.concatenate([dihedral, bond], 1)


def _quat(R):
    diag = jnp.diagonal(R, axis1=-2, axis2=-1)
    Rxx, Ryy, Rzz = diag[..., 0], diag[..., 1], diag[..., 2]
    mag = 0.5 * jnp.sqrt(jnp.abs(1 + jnp.stack(
        [Rxx - Ryy - Rzz, -Rxx + Ryy - Rzz, -Rxx - Ryy + Rzz], -1)) + 1e-12)
    signs = jnp.sign(jnp.stack(
        [R[:, 2, 1] - R[:, 1, 2], R[:, 0, 2] - R[:, 2, 0], R[:, 1, 0] - R[:, 0, 1]], -1))
    xyz = signs * mag
    w = jnp.sqrt(jax.nn.relu(1 + diag.sum(-1, keepdims=True)) + 1e-12) / 2.0
    return _norm(jnp.concatenate([xyz, w], -1))


def _dir_orient(X, edge_index):
    X_N, X_Ca, X_C = X[:, 0], X[:, 1], X[:, 2]
    u = _norm(X_Ca - X_N)
    v = _norm(X_C - X_Ca)
    b = _norm(u - v)
    n = _norm(jnp.cross(u, v))
    local_frame = jnp.stack([b, n, jnp.cross(b, n)], -1)
    node_j, node_i = edge_index[0], edge_index[1]
    t = _norm(X[:, jnp.array([0, 2, 3, 4])] - X_Ca[:, None, :])
    node_direction = jnp.matmul(t, local_frame).reshape(t.shape[0], -1)
    t = _norm(X[node_j] - X_Ca[node_i][:, None, :])
    e_ji = jnp.matmul(t, local_frame[node_i]).reshape(t.shape[0], -1)
    t = _norm(X[node_i] - X_Ca[node_j][:, None, :])
    e_ij = jnp.matmul(t, local_frame[node_j]).reshape(t.shape[0], -1)
    edge_direction = jnp.concatenate([e_ji, e_ij], -1)
    r = jnp.matmul(jnp.swapaxes(local_frame[node_i], -1, -2), local_frame[node_j])
    return node_direction, edge_direction, _quat(r)


def _rbf(D, D_min=0.0, D_max=20.0, D_count=16):
    mu = jnp.linspace(D_min, D_max, D_count).reshape(1, -1)
    sigma = (D_max - D_min) / D_count
    return jnp.exp(-((D[..., None] - mu) / sigma) ** 2)


def _safe_norm(x):
    return jnp.sqrt(jnp.sum(x * x, -1) + 1e-12)


def _dist_feat(X, edge_index):
    atoms = {'N': X[:, 0], 'Ca': X[:, 1], 'C': X[:, 2], 'O': X[:, 3], 'R': X[:, 4]}
    node_pairs = ['Ca-N', 'Ca-C', 'Ca-O', 'N-C', 'N-O', 'O-C', 'R-N', 'R-Ca', 'R-C', 'R-O']
    node_dist = jnp.concatenate(
        [_rbf(_safe_norm(atoms[p.split('-')[0]] - atoms[p.split('-')[1]])) for p in node_pairs], -1)
    names = ['N', 'Ca', 'C', 'O', 'R']
    edge_dist = jnp.concatenate(
        [_rbf(_safe_norm(atoms[a1][edge_index[0]] - atoms[a2][edge_index[1]]))
         for a1 in names for a2 in names], -1)
    return node_dist, edge_dist


def _pos_emb(edge_index, num_embeddings=16):
    d = (edge_index[0] - edge_index[1]).astype(_F32)
    freq = jnp.exp(jnp.arange(0, num_embeddings, 2, dtype=_F32) * (-np.log(10000.0) / num_embeddings))
    ang = d[:, None] * freq
    return jnp.concatenate([jnp.cos(ang), jnp.sin(ang)], -1)


def _geo_feat(X, edge_index):
    pe = _pos_emb(edge_index)
    na = _angle_feat(X)
    nd, ed = _dist_feat(X, edge_index)
    ndir, edir, eo = _dir_orient(X, edge_index)
    geo_node = jnp.concatenate([na, nd, ndir], -1)
    pe, eo, ed, edir = lax.optimization_barrier((pe, eo, ed, edir))
    geo_edge = jnp.concatenate([pe, eo, ed, edir], -1)
    return geo_node, geo_edge


def _layernorm(x, g, b):
    m = jnp.mean(x, axis=-1, keepdims=True)
    v = jnp.mean((x - m) ** 2, axis=-1, keepdims=True)
    return (x - m) * lax.rsqrt(v + 1e-5) * g + b


def _mm(a, b):
    return jnp.dot(a, b, preferred_element_type=_F32)


def _const_spec(shape):
    return pl.BlockSpec(shape, lambda i: tuple(0 for _ in shape))


def _node_init_body(hv, gn, w1, w2, bemb, gam, bet, wv, bv, out):
    t = _mm(hv[...], w1[...]) + _mm(gn[...], w2[...]) + bemb[...]
    t = t * (gam[...] * _BN_SCALE) + bet[...]
    out[...] = _mm(t, wv[...]) + bv[...]


def _node_init(hv, gn, w1, w2, bemb, gam, bet, wv, bv):
    return pl.pallas_call(
        _node_init_body,
        grid=(GN,),
        in_specs=[
            pl.BlockSpec((BN, HV_IN), lambda i: (i, 0)),
            pl.BlockSpec((BN, GEO_N), lambda i: (i, 0)),
            _const_spec((HV_IN, H)),
            _const_spec((GEO_N, H)),
            _const_spec((1, H)),
            _const_spec((1, H)),
            _const_spec((1, H)),
            _const_spec((H, H)),
            _const_spec((1, H)),
        ],
        out_specs=pl.BlockSpec((BN, H), lambda i: (i, 0)),
        out_shape=jax.ShapeDtypeStruct((N, H), _F32),
    )(hv, gn, w1, w2, bemb, gam, bet, wv, bv)


def _edge_init_body(ge, wemb, bemb, gam, bet, we, be, out):
    t = _mm(ge[...], wemb[...]) + bemb[...]
    t = t * (gam[...] * _BN_SCALE) + bet[...]
    out[...] = _mm(t, we[...]) + be[...]


def _edge_init(ge, wemb, bemb, gam, bet, we, be):
    return pl.pallas_call(
        _edge_init_body,
        grid=(GE,),
        in_specs=[
            pl.BlockSpec((BE, GEO_E), lambda i: (i, 0)),
            _const_spec((GEO_E, H)),
            _const_spec((1, H)),
            _const_spec((1, H)),
            _const_spec((1, H)),
            _const_spec((H, H)),
            _const_spec((1, H)),
        ],
        out_specs=pl.BlockSpec((BE, H), lambda i: (i, 0)),
        out_shape=jax.ShapeDtypeStruct((EP, H), _F32),
    )(ge, wemb, bemb, gam, bet, we, be)


def _qkv_body(x, wq, bq, wk, bk, wv, bv, q_out, kv_out):
    xx = x[...]
    q_out[...] = _mm(xx, wq[...]) + bq[...]
    kv_out[:, 0:H] = _mm(xx, wk[...]) + bk[...]
    kv_out[:, H:2 * H] = _mm(xx, wv[...]) + bv[...]


def _qkv(x, wq, bq, wk, bk, wv, bv):
    return pl.pallas_call(
        _qkv_body,
        grid=(GN,),
        in_specs=[
            pl.BlockSpec((BN, H), lambda i: (i, 0)),
            _const_spec((H, H)), _const_spec((1, H)),
            _const_spec((H, H)), _const_spec((1, H)),
            _const_spec((H, H)), _const_spec((1, H)),
        ],
        out_specs=[
            pl.BlockSpec((BN, H), lambda i: (i, 0)),
            pl.BlockSpec((BN, 2 * H), lambda i: (i, 0)),
        ],
        out_shape=[
            jax.ShapeDtypeStruct((N, H), _F32),
            jax.ShapeDtypeStruct((N, 2 * H), _F32),
        ],
    )(x, wq, bq, wk, bk, wv, bv)


def _attn_body(kv, qd, he, we, bwe, msg_out, a_out):
    e = _mm(he[...], we[...]) + bwe[...]
    ks = kv[:, 0:H] + e
    vs = kv[:, H:2 * H] + e
    m = qd[...] * ks
    i = pl.program_id(0)
    rows = i * BE + lax.broadcasted_iota(jnp.int32, (BE, 1), 0)
    msk = (rows < E).astype(_F32)
    for h in range(HEADS):
        sl = slice(h * DH, (h + 1) * DH)
        lg = jnp.sum(m[:, sl], axis=1, keepdims=True) * _ISQ
        a = jnp.exp(lg) * msk
        msg_out[:, sl] = a * vs[:, sl]
        a_out[:, h:h + 1] = a
    a_out[:, HEADS:H] = jnp.zeros((BE, H - HEADS), _F32)


def _attn_edges(kv_src, q_dst, he, we, bwe):
    return pl.pallas_call(
        _attn_body,
        grid=(GE,),
        in_specs=[
            pl.BlockSpec((BE, 2 * H), lambda i: (i, 0)),
            pl.BlockSpec((BE, H), lambda i: (i, 0)),
            pl.BlockSpec((BE, H), lambda i: (i, 0)),
            _const_spec((H, H)), _const_spec((1, H)),
        ],
        out_specs=[
            pl.BlockSpec((BE, H), lambda i: (i, 0)),
            pl.BlockSpec((BE, H), lambda i: (i, 0)),
        ],
        out_shape=[
            jax.ShapeDtypeStruct((EP, H), _F32),
            jax.ShapeDtypeStruct((EP, H), _F32),
        ],
    )(kv_src, q_dst, he, we, bwe)


def _nupd_body(pm, pa, hv, oh, g0, b0, w1, b1, w2, b2, g1, b1n,
               hmid, bsum, cnt):
    U = pm[0] + pm[1]
    S = pa[0, :, 0:HEADS] + pa[1, :, 0:HEADS]
    att = jnp.concatenate(
        [U[:, h * DH:(h + 1) * DH] / jnp.maximum(S[:, h:h + 1], 1e-16)
         for h in range(HEADS)], axis=1)
    h1 = _layernorm(hv[...] + att, g0[...], b0[...])
    r = jnp.maximum(_mm(h1, w1[...]) + b1[...], 0.0)
    dh2 = _mm(r, w2[...]) + b2[...]
    h2 = _layernorm(h1 + dh2, g1[...], b1n[...])
    hmid[...] = h2
    i = pl.program_id(0)

    @pl.when(i == 0)
    def _():
        bsum[...] = jnp.zeros_like(bsum)
        cnt[...] = jnp.zeros_like(cnt)

    ohv = oh[...]
    bsum[...] += lax.dot_general(ohv, h2, (((0,), (0,)), ((), ())),
                                 preferred_element_type=_F32)
    cnt[...] += lax.dot_general(ohv, jnp.ones((BN, H), _F32), (((0,), (0,)), ((), ())),
                                preferred_element_type=_F32)


def _node_update(parts_m, parts_a, hv, oh, g0, b0, w1, b1, w2, b2, g1, b1n):
    return pl.pallas_call(
        _nupd_body,
        grid=(GN,),
        in_specs=[
            pl.BlockSpec((2, BN, H), lambda i: (0, i, 0)),
            pl.BlockSpec((2, BN, H), lambda i: (0, i, 0)),
            pl.BlockSpec((BN, H), lambda i: (i, 0)),
            pl.BlockSpec((BN, NG), lambda i: (i, 0)),
            _const_spec((1, H)), _const_spec((1, H)),
            _const_spec((H, 4 * H)), _const_spec((1, 4 * H)),
            _const_spec((4 * H, H)), _const_spec((1, H)),
            _const_spec((1, H)), _const_spec((1, H)),
        ],
        out_specs=[
            pl.BlockSpec((BN, H), lambda i: (i, 0)),
            _const_spec((NG, H)),
            _const_spec((NG, H)),
        ],
        out_shape=[
            jax.ShapeDtypeStruct((N, H), _F32),
            jax.ShapeDtypeStruct((NG, H), _F32),
            jax.ShapeDtypeStruct((NG, H), _F32),
        ],
    )(parts_m, parts_a, hv, oh, g0, b0, w1, b1, w2, b2, g1, b1n)


def _gate_body(hmid, oh, bsum, cnt, c1, bc1, c2, bc2, out):
    cV = bsum[...] / jnp.maximum(cnt[...], 1.0)
    z = jnp.maximum(_mm(cV, c1[...]) + bc1[...], 0.0)
    g = 1.0 / (1.0 + jnp.exp(-(_mm(z, c2[...]) + bc2[...])))
    out[...] = hmid[...] * _mm(oh[...], g)


def _gate(hmid, oh, bsum, cnt, c1, bc1, c2, bc2):
    return pl.pallas_call(
        _gate_body,
        grid=(GN,),
        in_specs=[
            pl.BlockSpec((BN, H), lambda i: (i, 0)),
            pl.BlockSpec((BN, NG), lambda i: (i, 0)),
            _const_spec((NG, H)), _const_spec((NG, H)),
            _const_spec((H, H)), _const_spec((1, H)),
            _const_spec((H, H)), _const_spec((1, H)),
        ],
        out_specs=pl.BlockSpec((BN, H), lambda i: (i, 0)),
        out_shape=jax.ShapeDtypeStruct((N, H), _F32),
    )(hmid, oh, bsum, cnt, c1, bc1, c2, bc2)


def _emlp_body(hs, hd, he, wa, wb, wc, b11, w12, b12, gam, bet, out):
    hee = he[...]
    z = (_mm(hs[...], wa[...]) + _mm(hee, wb[...]) + _mm(hd[...], wc[...]) + b11[...])
    gelu = 0.5 * z * (1.0 + lax.erf(z * _ISQ2))
    msg = _mm(gelu, w12[...]) + b12[...]
    out[...] = (hee + msg) * (gam[...] * _BN_SCALE) + bet[...]


def _edge_mlp(hs, hd, he, wa, wb, wc, b11, w12, b12, gam, bet):
    return pl.pallas_call(
        _emlp_body,
        grid=(GE,),
        in_specs=[
            pl.BlockSpec((BE, H), lambda i: (i, 0)),
            pl.BlockSpec((BE, H), lambda i: (i, 0)),
            pl.BlockSpec((BE, H), lambda i: (i, 0)),
            _const_spec((H, H)), _const_spec((H, H)), _const_spec((H, H)),
            _const_spec((1, H)),
            _const_spec((H, H)), _const_spec((1, H)),
            _const_spec((1, H)), _const_spec((1, H)),
        ],
        out_specs=pl.BlockSpec((BE, H), lambda i: (i, 0)),
        out_shape=jax.ShapeDtypeStruct((EP, H), _F32),
    )(hs, hd, he, wa, wb, wc, b11, w12, b12, gam, bet)


_NW = 32
_PW = EP // _NW
_CH = 128
_NCH = _PW // _CH
_STRIPE = 640
_LAST = N - 15 * _STRIPE


def _sc_mesh():
    return plsc.VectorSubcoreMesh(core_axis_name="c", subcore_axis_name="s")


@functools.cache
def _make_sc_gather(D):
    @functools.partial(
        pl.kernel,
        mesh=_sc_mesh(),
        out_type=jax.ShapeDtypeStruct((EP, D), _F32),
        scratch_types=[
            pltpu.VMEM((_CH,), jnp.int32),
            pltpu.VMEM((_CH, D), _F32),
            pltpu.SemaphoreType.DMA,
        ],
    )
    def gk(table_hbm, idx_hbm, out_hbm, idx_v, rows_v, sem):
        c = lax.axis_index("c")
        s = lax.axis_index("s")
        base = (s * 2 + c) * _PW

        def body(j, carry):
            off = pl.multiple_of(base + j * _CH, _CH)
            pltpu.sync_copy(idx_hbm.at[pl.ds(off, _CH)], idx_v)
            pltpu.async_copy(table_hbm.at[idx_v], rows_v, sem).wait()
            pltpu.sync_copy(rows_v, out_hbm.at[pl.ds(off, _CH)])
            return carry

        lax.fori_loop(0, _NCH, body, 0)

    return gk


@functools.cache
def _make_sc_scatter():
    @functools.partial(
        pl.kernel,
        mesh=_sc_mesh(),
        out_type=jax.ShapeDtypeStruct((2, N, DV), _F32),
        scratch_types=[
            pltpu.VMEM((_CH,), jnp.int32),
            pltpu.VMEM((_CH, DV), _F32),
            pltpu.VMEM_SHARED((N, DV), _F32),
        ],
    )
    def sk(vals_hbm, idx_hbm, zeros_hbm, out_hbm, idx_v, rows_v, acc_sh):
        c = lax.axis_index("c")
        s = lax.axis_index("s")
        soff = pl.multiple_of(s * _STRIPE, 8)

        @pl.when(s < 15)
        def _():
            pltpu.sync_copy(zeros_hbm.at[pl.ds(soff, _STRIPE)],
                            acc_sh.at[pl.ds(soff, _STRIPE)])

        @pl.when(s == 15)
        def _():
            pltpu.sync_copy(zeros_hbm.at[pl.ds(15 * _STRIPE, _LAST)],
                            acc_sh.at[pl.ds(15 * _STRIPE, _LAST)])

        plsc.subcore_barrier()
        base = c * (EP // 2) + s * _PW

        def body(j, carry):
            off = pl.multiple_of(base + j * _CH, _CH)
            pltpu.sync_copy(idx_hbm.at[pl.ds(off, _CH)], idx_v)
            pltpu.sync_copy(vals_hbm.at[pl.ds(off, _CH)], rows_v)
            pltpu.sync_copy(rows_v, acc_sh.at[idx_v], add=True)
            return carry

        lax.fori_loop(0, _NCH, body, 0)
        plsc.subcore_barrier()

        @pl.when(s < 15)
        def _():
            pltpu.sync_copy(acc_sh.at[pl.ds(soff, _STRIPE)],
                            out_hbm.at[c, pl.ds(soff, _STRIPE)])

        @pl.when(s == 15)
        def _():
            pltpu.sync_copy(acc_sh.at[pl.ds(15 * _STRIPE, _LAST)],
                            out_hbm.at[c, pl.ds(15 * _STRIPE, _LAST)])

    return sk


def kernel_partial(X, h_V, edge_index, batch_id, params, nlayers):
    X = X.astype(_F32)
    h_V = h_V.astype(_F32)
    ei = edge_index.astype(jnp.int32)
    batch_id = batch_id.astype(jnp.int32)

    geo_node, geo_edge = _geo_feat(X, ei)
    pad = EP - E
    srcp = jnp.concatenate([ei[0], jnp.zeros((pad,), jnp.int32)])
    dstp = jnp.concatenate([ei[1], jnp.zeros((pad,), jnp.int32)])
    gep = jnp.pad(geo_edge, ((0, pad), (0, 0)))
    onehot = (batch_id[:, None] == jnp.arange(NG, dtype=jnp.int32)[None, :]).astype(_F32)
    zeros_acc = jnp.zeros((N, DV), _F32)

    p = params

    def T(lp):
        return jnp.transpose(lp['W'])

    def B(lp):
        return lp['b'][None, :]

    wembT = T(p['node_emb'])
    hV = _node_init(h_V, geo_node, wembT[:HV_IN], wembT[HV_IN:], B(p['node_emb']),
                    p['bn_n']['gamma'][None], p['bn_n']['beta'][None],
                    T(p['W_v']), B(p['W_v']))
    hE = _edge_init(gep, T(p['edge_emb']), B(p['edge_emb']),
                    p['bn_e']['gamma'][None], p['bn_e']['beta'][None],
                    T(p['W_e']), B(p['W_e']))

    gat128 = _make_sc_gather(H)
    gat256 = _make_sc_gather(2 * H)
    scat = _make_sc_scatter()

    for lp in p['layers'][:nlayers]:
        at = lp['attn']
        q, kv = _qkv(hV, T(at['Wq']), B(at['Wq']), T(at['Wk']), B(at['Wk']),
                     T(at['Wv']), B(at['Wv']))
        kv_src = gat256(kv, srcp)
        q_dst = gat128(q, dstp)
        edge_msg, edge_a = _attn_edges(kv_src, q_dst, hE, T(at['We']), B(at['We']))
        parts_m = scat(edge_msg, dstp, zeros_acc)
        parts_a = scat(edge_a, dstp, zeros_acc)
        hmid, bsum, cnt = _node_update(
            parts_m, parts_a, hV, onehot,
            lp['ln0']['gamma'][None], lp['ln0']['beta'][None],
            T(lp['ffn1']), B(lp['ffn1']), T(lp['ffn2']), B(lp['ffn2']),
            lp['ln1']['gamma'][None], lp['ln1']['beta'][None])
        hs = gat128(hmid, srcp)
        hd = gat128(hmid, dstp)
        w11T = T(lp['W11'])
        hE = _edge_mlp(hs, hd, hE,
                       w11T[0:H], w11T[H:2 * H], w11T[2 * H:3 * H], B(lp['W11']),
                       T(lp['W12']), B(lp['W12']),
                       lp['bn']['gamma'][None], lp['bn']['beta'][None])
        hV = _gate(hmid, onehot, bsum, cnt,
                   T(lp['ctx1']), B(lp['ctx1']), T(lp['ctx2']), B(lp['ctx2']))

    return hV


def kernel(X, h_V, edge_index, batch_id, params):
    return kernel_partial(X, h_V, edge_index, batch_id, params, 4)
